# baseline (speedup 1.0000x reference)
import jax
import jax.numpy as jnp
from jax.experimental import pallas as pl
from jax.experimental.pallas import tpu as pltpu

_B = 4096
_V = 100000
_NTOP = 1024
_BT = 8
_NB = _B // _BT


def _row_kernel(params_ref, tokens_ref, ids_ref, x_ref, out_ref):
    tile = x_ref[...]
    toks = tokens_ref[0, 0, :]

    m = jnp.max(tile, axis=1, keepdims=True)
    s = jnp.sum(jnp.exp(tile - m), axis=1)

    loc = jax.lax.broadcasted_iota(jnp.int32, (_BT, _V), 1)
    xt = jnp.sum(jnp.where(loc == toks[:, None], tile, 0.0), axis=1)

    ids = ids_ref[...]
    mask = jnp.any(toks[:, None] == ids[None, :], axis=1)

    g = jnp.exp(xt - m[:, 0]) / s
    w = jnp.where(mask, params_ref[2], params_ref[0])
    b = jnp.where(mask, params_ref[3], params_ref[1])
    out_ref[0, 0, :] = jax.nn.sigmoid(g * w + b)


def kernel(x, tokens, top_token_ids, gen_w, gen_b, top_w, top_b):
    tokens = tokens.astype(jnp.int32).reshape(_NB, 1, _BT)
    ids = top_token_ids.astype(jnp.int32)
    params = jnp.concatenate([gen_w.reshape(-1), gen_b.reshape(-1),
                              top_w.reshape(-1), top_b.reshape(-1)])
    out = pl.pallas_call(
        _row_kernel,
        grid=(_NB,),
        in_specs=[
            pl.BlockSpec(memory_space=pltpu.SMEM),
            pl.BlockSpec((1, 1, _BT), lambda i: (i, 0, 0)),
            pl.BlockSpec((_NTOP,), lambda i: (0,)),
            pl.BlockSpec((_BT, _V), lambda i: (i, 0)),
        ],
        out_specs=pl.BlockSpec((1, 1, _BT), lambda i: (i, 0, 0)),
        out_shape=jax.ShapeDtypeStruct((_NB, 1, _BT), jnp.float32),
        compiler_params=pltpu.CompilerParams(
            dimension_semantics=("arbitrary",)),
    )(params, tokens, ids, x)
    return out.reshape(_B)


# speedup vs baseline: 4.5634x; 4.5634x over previous
import jax
import jax.numpy as jnp
from jax.experimental import pallas as pl
from jax.experimental.pallas import tpu as pltpu

_B = 4096
_V = 100000
_NTOP = 1024
_VT = 1000
_NV = _V // _VT


def _col_kernel(params_ref, tokens_ref, ids_ref, x_ref, out_ref,
                m_ref, s_ref, xt_ref):
    j = pl.program_id(0)

    @pl.when(j == 0)
    def _():
        m_ref[...] = jnp.full((1, _B), -jnp.inf, jnp.float32)
        s_ref[...] = jnp.zeros((1, _B), jnp.float32)
        xt_ref[...] = jnp.zeros((1, _B), jnp.float32)

    tile = x_ref[...]
    toks = tokens_ref[...]

    loc = j * _VT + jax.lax.broadcasted_iota(jnp.int32, (_VT, _B), 0)
    xt_ref[...] += jnp.sum(jnp.where(loc == toks, tile, 0.0),
                           axis=0, keepdims=True)

    m_old = m_ref[...]
    m_new = jnp.maximum(m_old, jnp.max(tile, axis=0, keepdims=True))
    s_ref[...] = (s_ref[...] * jnp.exp(m_old - m_new)
                  + jnp.sum(jnp.exp(tile - m_new), axis=0, keepdims=True))
    m_ref[...] = m_new

    @pl.when(j == _NV - 1)
    def _():
        ids = ids_ref[...]
        mask = jnp.any(toks == ids, axis=0, keepdims=True)
        g = jnp.exp(xt_ref[...] - m_ref[...]) / s_ref[...]
        w = jnp.where(mask, params_ref[2], params_ref[0])
        b = jnp.where(mask, params_ref[3], params_ref[1])
        out_ref[...] = jax.nn.sigmoid(g * w + b)


def kernel(x, tokens, top_token_ids, gen_w, gen_b, top_w, top_b):
    xt_view = x.T
    tokens = tokens.astype(jnp.int32).reshape(1, _B)
    ids = top_token_ids.astype(jnp.int32).reshape(_NTOP, 1)
    params = jnp.concatenate([gen_w.reshape(-1), gen_b.reshape(-1),
                              top_w.reshape(-1), top_b.reshape(-1)])
    out = pl.pallas_call(
        _col_kernel,
        grid=(_NV,),
        in_specs=[
            pl.BlockSpec(memory_space=pltpu.SMEM),
            pl.BlockSpec((1, _B), lambda j: (0, 0)),
            pl.BlockSpec((_NTOP, 1), lambda j: (0, 0)),
            pl.BlockSpec((_VT, _B), lambda j: (j, 0)),
        ],
        out_specs=pl.BlockSpec((1, _B), lambda j: (0, 0)),
        out_shape=jax.ShapeDtypeStruct((1, _B), jnp.float32),
        scratch_shapes=[pltpu.VMEM((1, _B), jnp.float32),
                        pltpu.VMEM((1, _B), jnp.float32),
                        pltpu.VMEM((1, _B), jnp.float32)],
        compiler_params=pltpu.CompilerParams(
            dimension_semantics=("arbitrary",)),
    )(params, tokens, ids, xt_view)
    return out.reshape(_B)
